# baseline (device time: 77584 ns/iter reference)
import jax
import jax.numpy as jnp
from jax import lax
from jax.experimental import pallas as pl
from jax.experimental.pallas import tpu as pltpu

N_DEV = 4
B, H, D, BS = 16, 16, 64, 16
NB = 128
PAGES_LOCAL = 128
KEYS_LOCAL = PAGES_LOCAL * BS
NEG_INF = -1e30


def kernel(Q, K, V, bt, lens):
    lens2 = lens.reshape(B, 1).astype(jnp.int32)
    qh = Q.reshape(B, H, D).transpose(1, 0, 2)

    def body(q_ref, k_ref, v_ref, bt_ref, lens_ref, out_ref,
             kbuf, vbuf, comm_ref, copy_sems, send_sems, recv_sems):
        my_i = lax.axis_index("i")

        def stage(h, slot):
            ck = pltpu.make_async_copy(
                k_ref.at[:, :, h, :], kbuf.at[slot], copy_sems.at[slot, 0])
            cv = pltpu.make_async_copy(
                v_ref.at[:, :, h, :], vbuf.at[slot], copy_sems.at[slot, 1])
            ck.start()
            cv.start()
            return ck, cv

        pending = stage(0, 0)

        bt3 = bt_ref[:, :][:, :, None]
        lens3 = lens_ref[:, :][:, :, None]
        kpos = lax.broadcasted_iota(jnp.int32, (B, NB, 1), 1)
        pid = (lax.broadcasted_iota(jnp.int32, (1, 1, PAGES_LOCAL), 2)
               + my_i * PAGES_LOCAL)
        hit = (bt3 == pid) & (kpos < lens3)
        counts = jnp.sum(jnp.where(hit, 1.0, 0.0).astype(jnp.float32),
                         axis=1)
        cnt_keys = jnp.broadcast_to(
            counts[:, :, None], (B, PAGES_LOCAL, BS)
        ).reshape(B, KEYS_LOCAL)
        valid = cnt_keys > 0.0

        for h in range(H):
            slot = h % 2
            ck, cv = pending
            ck.wait()
            cv.wait()
            if h + 1 < H:
                pending = stage(h + 1, (h + 1) % 2)
            q_h = q_ref[h] * (D ** -0.5)
            k_h = kbuf[slot].reshape(KEYS_LOCAL, D)
            v_h = vbuf[slot].reshape(KEYS_LOCAL, D)
            s_h = lax.dot_general(
                q_h, k_h, (((1,), (1,)), ((), ())),
                preferred_element_type=jnp.float32,
            )
            m_h = jnp.max(jnp.where(valid, s_h, NEG_INF),
                          axis=1, keepdims=True)
            p_h = jnp.where(valid, cnt_keys * jnp.exp(s_h - m_h), 0.0)
            l_h = jnp.sum(p_h, axis=1, keepdims=True)
            o_h = lax.dot_general(
                p_h, v_h, (((1,), (0,)), ((), ())),
                preferred_element_type=jnp.float32,
            )
            comm_ref[0, h, :, 0:D] = o_h
            comm_ref[0, h, :, D:D + 1] = m_h
            comm_ref[0, h, :, D + 1:D + 2] = l_h

        barrier_sem = pltpu.get_barrier_semaphore()
        for t in range(1, N_DEV):
            pl.semaphore_signal(
                barrier_sem, inc=1,
                device_id=((my_i + t) % N_DEV,),
                device_id_type=pl.DeviceIdType.MESH,
            )
        pl.semaphore_wait(barrier_sem, N_DEV - 1)

        rdmas = []
        for t in range(1, N_DEV):
            rdma = pltpu.make_async_remote_copy(
                src_ref=comm_ref.at[0],
                dst_ref=comm_ref.at[t],
                send_sem=send_sems.at[t - 1],
                recv_sem=recv_sems.at[t - 1],
                device_id=((my_i + t) % N_DEV,),
                device_id_type=pl.DeviceIdType.MESH,
            )
            rdma.start()
            rdmas.append(rdma)

        acc_o = comm_ref[0, :, :, 0:D]
        acc_m = comm_ref[0, :, :, D:D + 1]
        acc_l = comm_ref[0, :, :, D + 1:D + 2]
        for t in range(1, N_DEV):
            rdmas[t - 1].wait()
            o_in = comm_ref[t, :, :, 0:D]
            m_in = comm_ref[t, :, :, D:D + 1]
            l_in = comm_ref[t, :, :, D + 1:D + 2]
            m_new = jnp.maximum(acc_m, m_in)
            a = jnp.exp(acc_m - m_new)
            bweight = jnp.exp(m_in - m_new)
            acc_o = acc_o * a + o_in * bweight
            acc_l = acc_l * a + l_in * bweight
            acc_m = m_new

        res = acc_o / acc_l
        out_ref[:, 0, :, :] = jnp.transpose(res, (1, 0, 2))

    return pl.pallas_call(
        body,
        out_shape=jax.ShapeDtypeStruct((B, 1, H, D), jnp.float32),
        in_specs=[
            pl.BlockSpec(memory_space=pltpu.VMEM),
            pl.BlockSpec(memory_space=pl.ANY),
            pl.BlockSpec(memory_space=pl.ANY),
            pl.BlockSpec(memory_space=pltpu.VMEM),
            pl.BlockSpec(memory_space=pltpu.VMEM),
        ],
        out_specs=pl.BlockSpec(memory_space=pltpu.VMEM),
        scratch_shapes=[
            pltpu.VMEM((2, PAGES_LOCAL, BS, D), jnp.float32),
            pltpu.VMEM((2, PAGES_LOCAL, BS, D), jnp.float32),
            pltpu.VMEM((N_DEV, H, B, PAGES_LOCAL), jnp.float32),
            pltpu.SemaphoreType.DMA((2, 2)),
            pltpu.SemaphoreType.DMA((N_DEV - 1,)),
            pltpu.SemaphoreType.DMA((N_DEV - 1,)),
        ],
        compiler_params=pltpu.CompilerParams(collective_id=0),
    )(qh, K, V, bt, lens2)


# device time: 18730 ns/iter; 4.1422x vs baseline; 4.1422x over previous
import jax
import jax.numpy as jnp
from jax import lax
from jax.experimental import pallas as pl
from jax.experimental.pallas import tpu as pltpu

N_DEV = 4
B, H, D, BS = 16, 16, 64, 16
NB = 128
PAGES_LOCAL = 128
NEG_INF = -1e30


def kernel(Q, K, V, bt, lens):
    kp = K.transpose(1, 2, 3, 0)
    vp = V.transpose(1, 2, 3, 0)

    def body(q_ref, k_ref, v_ref, bt_ref, lens_ref, out_ref,
             kwin, vwin, ktr, vtr, comm_ref, kv_sems, send_sems, recv_sems):
        my_i = lax.axis_index("i")

        HC = H // 4
        kv_copies = []
        for c in range(4):
            hs = slice(c * HC, (c + 1) * HC)
            ck = pltpu.make_async_copy(
                k_ref.at[:, hs, :, :], kwin.at[:, hs, :, :],
                kv_sems.at[c, 0])
            cv = pltpu.make_async_copy(
                v_ref.at[:, hs, :, :], vwin.at[:, hs, :, :],
                kv_sems.at[c, 1])
            ck.start()
            cv.start()
            kv_copies.append((ck, cv))

        bt3 = bt_ref[:, :][:, :, None]
        lens3 = lens_ref[:].reshape(B, 1, 1)
        kpos = lax.broadcasted_iota(jnp.int32, (B, NB, 1), 1)
        pid = (lax.broadcasted_iota(jnp.int32, (1, 1, PAGES_LOCAL), 2)
               + my_i * PAGES_LOCAL)
        hit = (bt3 == pid) & (kpos < lens3)
        counts = jnp.sum(jnp.where(hit, 1.0, 0.0).astype(jnp.float32),
                         axis=1)
        lc = jnp.where(counts > 0.0, jnp.log(counts), NEG_INF)
        lncnt = jnp.broadcast_to(
            lc[:, None, :], (B, BS, PAGES_LOCAL)
        ).reshape(B, BS * PAGES_LOCAL)

        barrier_sem = pltpu.get_barrier_semaphore()
        for t in range(1, N_DEV):
            pl.semaphore_signal(
                barrier_sem, inc=1,
                device_id=((my_i + t) % N_DEV,),
                device_id_type=pl.DeviceIdType.MESH,
            )
        pl.semaphore_wait(barrier_sem, N_DEV - 1)

        rdmas = []

        def send_half(half):
            hs = slice(half * (H // 2), (half + 1) * (H // 2))
            for t in range(1, N_DEV):
                rdma = pltpu.make_async_remote_copy(
                    src_ref=comm_ref.at[0, hs],
                    dst_ref=comm_ref.at[t, hs],
                    send_sem=send_sems.at[t - 1, half],
                    recv_sem=recv_sems.at[t - 1, half],
                    device_id=((my_i + t) % N_DEV,),
                    device_id_type=pl.DeviceIdType.MESH,
                )
                rdma.start()
                rdmas.append(rdma)

        KEYS = BS * PAGES_LOCAL
        for h in range(H):
            if h % HC == 0:
                ck, cv = kv_copies[h // HC]
                ck.wait()
                cv.wait()
            if h == H // 2:
                send_half(0)
            q_h = q_ref[:, 0, h, :] * (D ** -0.5)
            ktr[:] = jnp.transpose(
                kwin[:, h, :, :], (1, 0, 2)).reshape(D, KEYS)
            vtr[:] = jnp.transpose(
                vwin[:, h, :, :], (1, 0, 2)).reshape(D, KEYS)
            s_h = lax.dot_general(
                q_h, ktr[:], (((1,), (0,)), ((), ())),
                preferred_element_type=jnp.float32,
            ) + lncnt
            m_h = jnp.max(s_h, axis=1, keepdims=True)
            p_h = jnp.exp(s_h - m_h)
            l_h = jnp.sum(p_h, axis=1, keepdims=True)
            o_h = lax.dot_general(
                p_h, vtr[:], (((1,), (1,)), ((), ())),
                preferred_element_type=jnp.float32,
            )
            comm_ref[0, h, :, 0:D] = o_h
            comm_ref[0, h, :, D:D + 1] = m_h
            comm_ref[0, h, :, D + 1:D + 2] = l_h

        send_half(1)

        acc_o = comm_ref[0, :, :, 0:D]
        acc_m = comm_ref[0, :, :, D:D + 1]
        acc_l = comm_ref[0, :, :, D + 1:D + 2]
        for t in range(1, N_DEV):
            rdmas[t - 1].wait()
            rdmas[t + 2].wait()
            o_in = comm_ref[t, :, :, 0:D]
            m_in = comm_ref[t, :, :, D:D + 1]
            l_in = comm_ref[t, :, :, D + 1:D + 2]
            m_new = jnp.maximum(acc_m, m_in)
            a = jnp.exp(acc_m - m_new)
            bweight = jnp.exp(m_in - m_new)
            acc_o = acc_o * a + o_in * bweight
            acc_l = acc_l * a + l_in * bweight
            acc_m = m_new

        res = acc_o / acc_l
        out_ref[:, 0, :, :] = jnp.transpose(res, (1, 0, 2))

    return pl.pallas_call(
        body,
        out_shape=jax.ShapeDtypeStruct((B, 1, H, D), jnp.float32),
        in_specs=[
            pl.BlockSpec(memory_space=pltpu.VMEM),
            pl.BlockSpec(memory_space=pl.ANY),
            pl.BlockSpec(memory_space=pl.ANY),
            pl.BlockSpec(memory_space=pltpu.VMEM),
            pl.BlockSpec(memory_space=pltpu.VMEM),
        ],
        out_specs=pl.BlockSpec(memory_space=pltpu.VMEM),
        scratch_shapes=[
            pltpu.VMEM((BS, H, D, PAGES_LOCAL), jnp.float32),
            pltpu.VMEM((BS, H, D, PAGES_LOCAL), jnp.float32),
            pltpu.VMEM((D, BS * PAGES_LOCAL), jnp.float32),
            pltpu.VMEM((D, BS * PAGES_LOCAL), jnp.float32),
            pltpu.VMEM((N_DEV, H, B, PAGES_LOCAL), jnp.float32),
            pltpu.SemaphoreType.DMA((4, 2)),
            pltpu.SemaphoreType.DMA((N_DEV - 1, 2)),
            pltpu.SemaphoreType.DMA((N_DEV - 1, 2)),
        ],
        compiler_params=pltpu.CompilerParams(
            collective_id=0, vmem_limit_bytes=60 * 1024 * 1024),
    )(Q, kp, vp, bt, lens)
